# baseline (speedup 1.0000x reference)
import functools

import jax
import jax.numpy as jnp
from jax import lax
from jax.experimental import pallas as pl
from jax.experimental.pallas import tpu as pltpu
from jax.experimental.pallas import tpu_sc as plsc

L = 16
NW = 32
C = 8192


@functools.lru_cache(maxsize=None)
def _build(n, res):
    per_w = n // NW
    n_chunks = per_w // C
    assert n_chunks >= 4 and n_chunks % 2 == 0
    mesh = plsc.VectorSubcoreMesh(core_axis_name="c", subcore_axis_name="s")

    buf = lambda dt: pltpu.VMEM((C,), dt)

    @functools.partial(
        pl.kernel,
        out_type=jax.ShapeDtypeStruct((n,), jnp.float32),
        mesh=mesh,
        scratch_types=[
            buf(jnp.float32), buf(jnp.int32), buf(jnp.float32),
            buf(jnp.float32), buf(jnp.float32),
            buf(jnp.float32), buf(jnp.int32), buf(jnp.float32),
            buf(jnp.float32), buf(jnp.float32),
            buf(jnp.float32),
            pltpu.SemaphoreType.DMA,
            pltpu.SemaphoreType.DMA,
            pltpu.SemaphoreType.DMA,
        ],
    )
    def grid_lookup(inp_hbm, tab_hbm, tab1_hbm, out_hbm,
                    xA, idxA, tA, loA, hiA, xB, idxB, tB, loB, hiB,
                    o_v, sem_in, semA, semB):
        wid = lax.axis_index("s") * 2 + lax.axis_index("c")
        w_base = wid * per_w

        A = (xA, idxA, tA, loA, hiA, semA)
        B = (xB, idxB, tB, loB, hiB, semB)

        def in_start(g, bufs):
            pltpu.async_copy(inp_hbm.at[pl.ds(w_base + g * C, C)],
                             bufs[0], sem_in)

        def in_wait():
            pltpu.make_async_copy(inp_hbm.at[pl.ds(0, C)], xA, sem_in).wait()

        def prep(bufs):
            x_v, idx_v, t_v = bufs[0], bufs[1], bufs[2]

            def body(i, _):
                s = pl.ds(i * L, L)
                scaled = x_v[s] * float(res - 1)
                low = jnp.clip(scaled.astype(jnp.int32), 0, res - 2)
                idx_v[s] = low
                t_v[s] = scaled - low.astype(jnp.float32)
                return 0

            lax.fori_loop(0, C // L, body, 0, unroll=4)

        def fire(bufs):
            idx_v, lo_v, hi_v, sem = bufs[1], bufs[3], bufs[4], bufs[5]
            pltpu.async_copy(tab_hbm.at[idx_v], lo_v, sem)
            pltpu.async_copy(tab1_hbm.at[idx_v], hi_v, sem)

        def drain(bufs):
            lo_v, hi_v, sem = bufs[3], bufs[4], bufs[5]
            pltpu.make_async_copy(tab_hbm.at[pl.ds(0, C)], lo_v, sem).wait()
            pltpu.make_async_copy(tab_hbm.at[pl.ds(0, C)], hi_v, sem).wait()

        def lerp_out(g, bufs):
            t_v, lo_v, hi_v = bufs[2], bufs[3], bufs[4]

            def body(i, _):
                s = pl.ds(i * L, L)
                t = t_v[s]
                o_v[s] = lo_v[s] * (1.0 - t) + hi_v[s] * t
                return 0

            lax.fori_loop(0, C // L, body, 0, unroll=4)
            pltpu.sync_copy(o_v, out_hbm.at[pl.ds(w_base + g * C, C)])

        last = n_chunks - 1

        in_start(0, A)
        in_wait()
        prep(A)
        fire(A)
        in_start(1, B)
        in_wait()
        prep(B)
        fire(B)
        in_start(2, A)
        drain(A)
        lerp_out(0, A)

        def pair_body(gg, _):
            g = 2 * gg + 2
            in_wait()
            prep(A)
            fire(A)
            in_start(jnp.minimum(g + 1, last), B)
            drain(B)
            lerp_out(g - 1, B)

            g2 = g + 1
            in_wait()
            prep(B)
            fire(B)
            in_start(jnp.minimum(g2 + 1, last), A)
            drain(A)
            lerp_out(g2 - 1, A)
            return 0

        lax.fori_loop(0, (n_chunks - 2) // 2, pair_body, 0)

        drain(B)
        lerp_out(last, B)
        in_wait()

    return grid_lookup


def kernel(input, feature_params):
    return _build(input.shape[0], feature_params.shape[0])(
        input, feature_params, feature_params[1:])


# speedup vs baseline: 1.7883x; 1.7883x over previous
import functools

import jax
import jax.numpy as jnp
from jax import lax
from jax.experimental import pallas as pl
from jax.experimental.pallas import tpu as pltpu
from jax.experimental.pallas import tpu_sc as plsc

L = 16
NW = 32
C = 4096


@functools.lru_cache(maxsize=None)
def _build(n, res):
    per_w = n // NW
    n_chunks = per_w // C
    assert n_chunks >= 4 and n_chunks % 2 == 0
    mesh = plsc.VectorSubcoreMesh(core_axis_name="c", subcore_axis_name="s")

    buf = lambda dt: pltpu.VMEM((C,), dt)

    @functools.partial(
        pl.kernel,
        out_type=jax.ShapeDtypeStruct((n,), jnp.float32),
        mesh=mesh,
        scratch_types=[
            buf(jnp.float32), buf(jnp.int32), buf(jnp.int32),
            buf(jnp.float32),
            buf(jnp.float32), buf(jnp.float32),
            buf(jnp.float32), buf(jnp.int32), buf(jnp.int32),
            buf(jnp.float32),
            buf(jnp.float32), buf(jnp.float32),
            buf(jnp.float32),
            pltpu.VMEM_SHARED((res,), jnp.float32),
            pltpu.SemaphoreType.DMA,
            pltpu.SemaphoreType.DMA,
            pltpu.SemaphoreType.DMA,
        ],
    )
    def grid_lookup(inp_hbm, tab_hbm, tab1_hbm, out_hbm,
                    xA, idxA, ihA, tA, loA, hiA, xB, idxB, ihB, tB, loB, hiB,
                    o_v, stab, sem_in, semA, semB):
        sid = lax.axis_index("s")
        wid = sid * 2 + lax.axis_index("c")
        w_base = wid * per_w

        A = (xA, idxA, ihA, tA, loA, hiA, semA)
        B = (xB, idxB, ihB, tB, loB, hiB, semB)

        seg = (res // 16 + 7) // 8 * 8
        rem = res - seg * 15
        n_full = rem // C
        tails = (seg - n_full * C, rem - n_full * C)

        def stage_piece(off, size, bounce):
            pltpu.sync_copy(tab_hbm.at[pl.ds(off, size)],
                            bounce.at[pl.ds(0, size)])
            pltpu.sync_copy(bounce.at[pl.ds(0, size)],
                            stab.at[pl.ds(off, size)])

        for j in range(n_full):
            stage_piece(sid * seg + j * C, C, xA)

        @pl.when(sid < 15)
        def _():
            stage_piece(sid * seg + n_full * C, tails[0], xB)

        @pl.when(sid == 15)
        def _():
            stage_piece(15 * seg + n_full * C, tails[1], loA)

        plsc.subcore_barrier()

        def in_start(g, bufs):
            pltpu.async_copy(inp_hbm.at[pl.ds(w_base + g * C, C)],
                             bufs[0], sem_in)

        def in_wait():
            pltpu.make_async_copy(inp_hbm.at[pl.ds(0, C)], xA, sem_in).wait()

        def prep(bufs):
            x_v, idx_v, ih_v, t_v = bufs[0], bufs[1], bufs[2], bufs[3]

            def body(i, _):
                s = pl.ds(i * L, L)
                scaled = x_v[s] * float(res - 1)
                low = jnp.clip(scaled.astype(jnp.int32), 0, res - 2)
                idx_v[s] = low
                ih_v[s] = low + 1
                t_v[s] = scaled - low.astype(jnp.float32)
                return 0

            lax.fori_loop(0, C // L, body, 0, unroll=4)

        def fire(bufs):
            idx_v, ih_v, lo_v, hi_v, sem = (bufs[1], bufs[2], bufs[4],
                                            bufs[5], bufs[6])
            pltpu.async_copy(stab.at[idx_v], lo_v, sem)
            pltpu.async_copy(stab.at[ih_v], hi_v, sem)

        def drain(bufs):
            lo_v, hi_v, sem = bufs[4], bufs[5], bufs[6]
            pltpu.make_async_copy(tab_hbm.at[pl.ds(0, C)], lo_v, sem).wait()
            pltpu.make_async_copy(tab_hbm.at[pl.ds(0, C)], hi_v, sem).wait()

        def lerp_out(g, bufs):
            t_v, lo_v, hi_v = bufs[3], bufs[4], bufs[5]

            def body(i, _):
                s = pl.ds(i * L, L)
                t = t_v[s]
                o_v[s] = lo_v[s] * (1.0 - t) + hi_v[s] * t
                return 0

            lax.fori_loop(0, C // L, body, 0, unroll=4)
            pltpu.sync_copy(o_v, out_hbm.at[pl.ds(w_base + g * C, C)])

        last = n_chunks - 1

        in_start(0, A)
        in_wait()
        prep(A)
        fire(A)
        in_start(1, B)
        in_wait()
        prep(B)
        fire(B)
        in_start(2, A)
        drain(A)
        lerp_out(0, A)

        def pair_body(gg, _):
            g = 2 * gg + 2
            in_wait()
            prep(A)
            fire(A)
            in_start(jnp.minimum(g + 1, last), B)
            drain(B)
            lerp_out(g - 1, B)

            g2 = g + 1
            in_wait()
            prep(B)
            fire(B)
            in_start(jnp.minimum(g2 + 1, last), A)
            drain(A)
            lerp_out(g2 - 1, A)
            return 0

        lax.fori_loop(0, (n_chunks - 2) // 2, pair_body, 0)

        drain(B)
        lerp_out(last, B)
        in_wait()

    return grid_lookup


def kernel(input, feature_params):
    return _build(input.shape[0], feature_params.shape[0])(
        input, feature_params, feature_params[1:])
